# baseline (speedup 1.0000x reference)
import functools

import jax
import jax.numpy as jnp
from jax import lax
from jax.experimental import pallas as pl
from jax.experimental.pallas import tpu as pltpu
from jax.experimental.pallas import tpu_sc as plsc

_SPARSITY = 0.15
_GAIN = 3.0
_LEAK = 0.1

_NC = 2
_NS = 16
_L = 16
_NBINS = 256
_NVREG_HIST = _NBINS // _L


def _find_bucket(hists, kk):
    iota = lax.iota(jnp.int32, _L)
    carry = jnp.int32(0)
    cs = [None] * _NVREG_HIST
    for j in range(_NVREG_HIST - 1, -1, -1):
        h = hists[j]
        sfx = lax.rev(plsc.cumsum(lax.rev(h, (0,))), (0,))
        cs[j] = sfx + carry
        carry = carry + jnp.sum(h)
    bucket = jnp.int32(-1)
    for j in range(_NVREG_HIST):
        ib = iota + jnp.int32(j * _L)
        cand = jnp.where(cs[j] >= kk, ib, jnp.int32(-1))
        bucket = jnp.maximum(bucket, jnp.max(cand))
    c_at = jnp.int32(0)
    h_at = jnp.int32(0)
    for j in range(_NVREG_HIST):
        ib = iota + jnp.int32(j * _L)
        hit = ib == bucket
        c_at = c_at + jnp.sum(jnp.where(hit, cs[j], jnp.int32(0)))
        h_at = h_at + jnp.sum(jnp.where(hit, hists[j], jnp.int32(0)))
    n_above = c_at - h_at
    return bucket, kk - n_above


def _clear_hist(hist_ref):
    zeros = jnp.zeros((_L,), jnp.int32)

    @plsc.parallel_loop(0, _NBINS * _L // _L, unroll=8)
    def _clr(i):
        hist_ref[pl.ds(i * _L, _L)] = zeros


def _merge_hist(hist_ref):
    def mbody(r, c):
        for j in range(_NVREG_HIST):
            acc = (hist_ref[pl.ds(j * _L, _L)]
                   + hist_ref[pl.ds(r * _NBINS + j * _L, _L)])
            hist_ref[pl.ds(j * _L, _L)] = acc
        return c

    lax.fori_loop(1, _L, mbody, 0)


def _read_hist(hist_ref):
    return [hist_ref[pl.ds(j * _L, _L)] for j in range(_NVREG_HIST)]


def kernel(x):
    rows, n = x.shape
    k = max(int(n * _SPARSITY), 1)
    nw = _NC * _NS
    rows_per_w = rows // nw
    nvec = n // _L
    mesh = plsc.VectorSubcoreMesh(core_axis_name="c", subcore_axis_name="s")

    @functools.partial(
        pl.kernel,
        out_type=jax.ShapeDtypeStruct((rows, n), jnp.float32),
        mesh=mesh,
        compiler_params=pltpu.CompilerParams(needs_layout_passes=False),
        scratch_types=[
            pltpu.VMEM((n,), jnp.float32),
            pltpu.VMEM((n,), jnp.uint32),
            pltpu.VMEM((n,), jnp.float32),
            pltpu.VMEM((_NBINS * _L,), jnp.int32),
        ],
    )
    def sc_topk_mask(x_hbm, out_hbm, xbuf, keys, obuf, hist):
        wid = lax.axis_index("s") * _NC + lax.axis_index("c")
        laneoff = lax.iota(jnp.int32, _L) * jnp.int32(_NBINS)
        ones = jnp.ones((_L,), jnp.int32)

        def row_body(r, carry_unused):
            row = wid * rows_per_w + r
            pltpu.sync_copy(x_hbm.at[row], xbuf)

            _clear_hist(hist)

            @plsc.parallel_loop(0, nvec, unroll=8)
            def _pass0(i):
                v = xbuf[pl.ds(i * _L, _L)]
                b = lax.bitcast_convert_type(v, jnp.uint32)
                neg = (b >> jnp.uint32(31)) != jnp.uint32(0)
                key = jnp.where(neg, ~b, b | jnp.uint32(0x80000000))
                keys[pl.ds(i * _L, _L)] = key
                idx = (key >> jnp.uint32(24)).astype(jnp.int32) + laneoff
                plsc.addupdate_scatter(hist, [idx], ones)
            _merge_hist(hist)
            bucket, kk = _find_bucket(_read_hist(hist), jnp.int32(k))
            prefix = bucket.astype(jnp.uint32)

            for lvl in range(1, 4):
                shift_b = jnp.uint32(24 - 8 * lvl)
                shift_p = jnp.uint32(32 - 8 * lvl)
                _clear_hist(hist)
                pfx = prefix

                @plsc.parallel_loop(0, nvec, unroll=8)
                def _passl(i, shift_b=shift_b, shift_p=shift_p, pfx=pfx):
                    key = keys[pl.ds(i * _L, _L)]
                    match = (key >> shift_p) == pfx
                    idx = ((key >> shift_b) & jnp.uint32(0xFF)).astype(
                        jnp.int32) + laneoff
                    plsc.addupdate_scatter(hist, [idx], ones, mask=match)
                _merge_hist(hist)
                bucket, kk = _find_bucket(_read_hist(hist), kk)
                prefix = (prefix << jnp.uint32(8)) | bucket.astype(jnp.uint32)

            tvec = jnp.full((_L,), prefix, dtype=jnp.uint32)
            tneg = tvec < jnp.uint32(0x80000000)
            tbits = jnp.where(tneg, ~tvec, tvec ^ jnp.uint32(0x80000000))
            thr = lax.bitcast_convert_type(tbits, jnp.float32)

            @plsc.parallel_loop(0, nvec, unroll=8)
            def _passo(i):
                v = xbuf[pl.ds(i * _L, _L)]
                m = jnp.where(v >= thr, jnp.float32(1.0), jnp.float32(_LEAK))
                obuf[pl.ds(i * _L, _L)] = v * m * jnp.float32(_GAIN)
            pltpu.sync_copy(obuf, out_hbm.at[row])
            return carry_unused

        lax.fori_loop(0, rows_per_w, row_body, 0)

    return sc_topk_mask(x)


# speedup vs baseline: 1.0522x; 1.0522x over previous
import functools

import jax
import jax.numpy as jnp
from jax import lax
from jax.experimental import pallas as pl
from jax.experimental.pallas import tpu as pltpu
from jax.experimental.pallas import tpu_sc as plsc

_SPARSITY = 0.15
_GAIN = 3.0
_LEAK = 0.1

_NC = 2
_NS = 16
_L = 16
_NBINS = 256
_NVREG_HIST = _NBINS // _L
_RSTRIDE = 257


def _find_bucket(hists, kk):
    iota = lax.iota(jnp.int32, _L)
    carry = jnp.int32(0)
    cs = [None] * _NVREG_HIST
    for j in range(_NVREG_HIST - 1, -1, -1):
        h = hists[j]
        sfx = lax.rev(plsc.cumsum(lax.rev(h, (0,))), (0,))
        cs[j] = sfx + carry
        carry = carry + jnp.sum(h)
    bucket = jnp.int32(-1)
    for j in range(_NVREG_HIST):
        ib = iota + jnp.int32(j * _L)
        cand = jnp.where(cs[j] >= kk, ib, jnp.int32(-1))
        bucket = jnp.maximum(bucket, jnp.max(cand))
    c_at = jnp.int32(0)
    h_at = jnp.int32(0)
    for j in range(_NVREG_HIST):
        ib = iota + jnp.int32(j * _L)
        hit = ib == bucket
        c_at = c_at + jnp.sum(jnp.where(hit, cs[j], jnp.int32(0)))
        h_at = h_at + jnp.sum(jnp.where(hit, hists[j], jnp.int32(0)))
    n_above = c_at - h_at
    return bucket, kk - n_above


def _clear_hist(hist_ref):
    zeros = jnp.zeros((_L,), jnp.int32)

    @plsc.parallel_loop(0, _RSTRIDE * _L // _L, unroll=8)
    def _clr(i):
        hist_ref[pl.ds(i * _L, _L)] = zeros


def _merge_hist(hist_ref):
    def mbody(r, c):
        for j in range(_NVREG_HIST):
            acc = (hist_ref[pl.ds(j * _L, _L)]
                   + hist_ref[pl.ds(r * _RSTRIDE + j * _L, _L)])
            hist_ref[pl.ds(j * _L, _L)] = acc
        return c

    lax.fori_loop(1, _L, mbody, 0)


def _read_hist(hist_ref):
    return [hist_ref[pl.ds(j * _L, _L)] for j in range(_NVREG_HIST)]


def kernel(x):
    rows, n = x.shape
    k = max(int(n * _SPARSITY), 1)
    nw = _NC * _NS
    rows_per_w = rows // nw
    nvec = n // _L
    mesh = plsc.VectorSubcoreMesh(core_axis_name="c", subcore_axis_name="s")

    @functools.partial(
        pl.kernel,
        out_type=jax.ShapeDtypeStruct((rows, n), jnp.float32),
        mesh=mesh,
        compiler_params=pltpu.CompilerParams(needs_layout_passes=False),
        scratch_types=[
            pltpu.VMEM((n,), jnp.float32),
            pltpu.VMEM((n,), jnp.uint32),
            pltpu.VMEM((n,), jnp.float32),
            pltpu.VMEM((_RSTRIDE * _L,), jnp.int32),
        ],
    )
    def sc_topk_mask(x_hbm, out_hbm, xbuf, keys, obuf, hist):
        wid = lax.axis_index("s") * _NC + lax.axis_index("c")
        laneoff = lax.iota(jnp.int32, _L) * jnp.int32(_RSTRIDE)
        ones = jnp.ones((_L,), jnp.int32)

        def row_body(r, carry_unused):
            row = wid * rows_per_w + r
            pltpu.sync_copy(x_hbm.at[row], xbuf)

            _clear_hist(hist)

            @plsc.parallel_loop(0, nvec, unroll=8)
            def _pass0(i):
                v = xbuf[pl.ds(i * _L, _L)]
                b = lax.bitcast_convert_type(v, jnp.uint32)
                neg = (b >> jnp.uint32(31)) != jnp.uint32(0)
                key = jnp.where(neg, ~b, b | jnp.uint32(0x80000000))
                keys[pl.ds(i * _L, _L)] = key
                idx = (key >> jnp.uint32(24)).astype(jnp.int32) + laneoff
                plsc.addupdate_scatter(hist, [idx], ones)
            _merge_hist(hist)
            bucket, kk = _find_bucket(_read_hist(hist), jnp.int32(k))
            prefix = bucket.astype(jnp.uint32)

            for lvl in range(1, 4):
                shift_b = jnp.uint32(24 - 8 * lvl)
                shift_p = jnp.uint32(32 - 8 * lvl)
                _clear_hist(hist)
                pfx = prefix

                @plsc.parallel_loop(0, nvec, unroll=8)
                def _passl(i, shift_b=shift_b, shift_p=shift_p, pfx=pfx):
                    key = keys[pl.ds(i * _L, _L)]
                    match = (key >> shift_p) == pfx
                    idx = ((key >> shift_b) & jnp.uint32(0xFF)).astype(
                        jnp.int32) + laneoff
                    plsc.addupdate_scatter(hist, [idx], ones, mask=match)
                _merge_hist(hist)
                bucket, kk = _find_bucket(_read_hist(hist), kk)
                prefix = (prefix << jnp.uint32(8)) | bucket.astype(jnp.uint32)

            tvec = jnp.full((_L,), prefix, dtype=jnp.uint32)
            tneg = tvec < jnp.uint32(0x80000000)
            tbits = jnp.where(tneg, ~tvec, tvec ^ jnp.uint32(0x80000000))
            thr = lax.bitcast_convert_type(tbits, jnp.float32)

            @plsc.parallel_loop(0, nvec, unroll=8)
            def _passo(i):
                v = xbuf[pl.ds(i * _L, _L)]
                m = jnp.where(v >= thr, jnp.float32(1.0), jnp.float32(_LEAK))
                obuf[pl.ds(i * _L, _L)] = v * m * jnp.float32(_GAIN)
            pltpu.sync_copy(obuf, out_hbm.at[row])
            return carry_unused

        lax.fori_loop(0, rows_per_w, row_body, 0)

    return sc_topk_mask(x)


# speedup vs baseline: 1.3333x; 1.2671x over previous
import functools

import jax
import jax.numpy as jnp
from jax import lax
from jax.experimental import pallas as pl
from jax.experimental.pallas import tpu as pltpu
from jax.experimental.pallas import tpu_sc as plsc

_SPARSITY = 0.15
_GAIN = 3.0
_LEAK = 0.1

_NC = 2
_NS = 16
_L = 16
_NBINS = 256
_NVREG_HIST = _NBINS // _L


def _find_bucket(hists, kk):
    iota = lax.iota(jnp.int32, _L)
    carry = jnp.int32(0)
    cs = [None] * _NVREG_HIST
    for j in range(_NVREG_HIST - 1, -1, -1):
        h = hists[j]
        sfx = lax.rev(plsc.cumsum(lax.rev(h, (0,))), (0,))
        cs[j] = sfx + carry
        carry = carry + jnp.sum(h)
    bucket = jnp.int32(-1)
    for j in range(_NVREG_HIST):
        ib = iota + jnp.int32(j * _L)
        cand = jnp.where(cs[j] >= kk, ib, jnp.int32(-1))
        bucket = jnp.maximum(bucket, jnp.max(cand))
    c_at = jnp.int32(0)
    h_at = jnp.int32(0)
    for j in range(_NVREG_HIST):
        ib = iota + jnp.int32(j * _L)
        hit = ib == bucket
        c_at = c_at + jnp.sum(jnp.where(hit, cs[j], jnp.int32(0)))
        h_at = h_at + jnp.sum(jnp.where(hit, hists[j], jnp.int32(0)))
    n_above = c_at - h_at
    return bucket, kk - n_above


def _clear_hist(hist_ref):
    zeros = jnp.zeros((_L,), jnp.int32)
    for j in range(_NVREG_HIST):
        hist_ref[pl.ds(j * _L, _L)] = zeros


def _read_hist(hist_ref):
    return [hist_ref[pl.ds(j * _L, _L)] for j in range(_NVREG_HIST)]


def kernel(x):
    rows, n = x.shape
    k = max(int(n * _SPARSITY), 1)
    nw = _NC * _NS
    rows_per_w = rows // nw
    nvec = n // _L
    mesh = plsc.VectorSubcoreMesh(core_axis_name="c", subcore_axis_name="s")

    @functools.partial(
        pl.kernel,
        out_type=jax.ShapeDtypeStruct((rows, n), jnp.float32),
        mesh=mesh,
        compiler_params=pltpu.CompilerParams(needs_layout_passes=False),
        scratch_types=[
            pltpu.VMEM((n,), jnp.float32),
            pltpu.VMEM((n,), jnp.float32),
            pltpu.VMEM((n + _L,), jnp.int32),
            pltpu.VMEM((_NBINS,), jnp.int32),
        ],
    )
    def sc_topk_mask(x_hbm, out_hbm, xbuf, kbuf, cbuf, hist):
        wid = lax.axis_index("s") * _NC + lax.axis_index("c")

        def row_body(r, carry_unused):
            row = wid * rows_per_w + r
            pltpu.sync_copy(x_hbm.at[row], xbuf)

            _clear_hist(hist)

            @plsc.parallel_loop(0, nvec, unroll=8)
            def _pass0(i):
                v = xbuf[pl.ds(i * _L, _L)]
                b = lax.bitcast_convert_type(v, jnp.uint32)
                neg = (b >> jnp.uint32(31)) != jnp.uint32(0)
                key = jnp.where(neg, ~b, b | jnp.uint32(0x80000000))
                kbuf[pl.ds(i * _L, _L)] = lax.bitcast_convert_type(
                    key, jnp.float32)
                bucket = (key >> jnp.uint32(24)).astype(jnp.int32)
                cnt, last = plsc.scan_count(bucket)
                plsc.addupdate_scatter(hist, [bucket], cnt, mask=last)

            bucket, kk = _find_bucket(_read_hist(hist), jnp.int32(k))
            pfx8 = bucket.astype(jnp.uint32)

            _clear_hist(hist)
            off0 = jnp.zeros((_L,), jnp.int32)

            @plsc.parallel_loop(0, nvec, unroll=8, carry=off0)
            def _pass1(i, off):
                key = lax.bitcast_convert_type(kbuf[pl.ds(i * _L, _L)],
                                               jnp.uint32)
                match = (key >> jnp.uint32(24)) == pfx8
                bucket = ((key >> jnp.uint32(16))
                          & jnp.uint32(0xFF)).astype(jnp.int32)
                cnt, last = plsc.scan_count(bucket, mask=match)
                plsc.addupdate_scatter(hist, [bucket], cnt,
                                       mask=last & match)
                csum = plsc.cumsum(match.astype(jnp.int32))
                idx = off + csum - jnp.int32(1)
                plsc.store_scatter(
                    cbuf, [idx],
                    lax.bitcast_convert_type(key, jnp.int32), mask=match)
                return off + plsc.all_reduce_population_count(match)

            off_final = _pass1
            bucket, kk = _find_bucket(_read_hist(hist), kk)
            prefix = (pfx8 << jnp.uint32(8)) | bucket.astype(jnp.uint32)

            cnt_sc = jnp.max(off_final)
            padkey = (~prefix) << jnp.uint32(16)
            plsc.store_scatter(
                cbuf, [off_final + lax.iota(jnp.int32, _L)],
                lax.bitcast_convert_type(
                    jnp.full((_L,), padkey, dtype=jnp.uint32), jnp.int32))
            ct = lax.shift_right_logical(cnt_sc + jnp.int32(15), 4)

            for lvl in (2, 3):
                shift_b = jnp.uint32(24 - 8 * lvl)
                shift_p = jnp.uint32(32 - 8 * lvl)
                _clear_hist(hist)
                pfx = prefix

                @plsc.parallel_loop(0, ct, unroll=4)
                def _passc(i, shift_b=shift_b, shift_p=shift_p, pfx=pfx):
                    key = lax.bitcast_convert_type(cbuf[pl.ds(i * _L, _L)],
                                                   jnp.uint32)
                    match = (key >> shift_p) == pfx
                    bucket = ((key >> shift_b)
                              & jnp.uint32(0xFF)).astype(jnp.int32)
                    cnt, last = plsc.scan_count(bucket, mask=match)
                    plsc.addupdate_scatter(hist, [bucket], cnt,
                                           mask=last & match)

                bucket, kk = _find_bucket(_read_hist(hist), kk)
                prefix = (prefix << jnp.uint32(8)) | bucket.astype(jnp.uint32)

            tvec = jnp.full((_L,), prefix, dtype=jnp.uint32)
            tneg = tvec < jnp.uint32(0x80000000)
            tbits = jnp.where(tneg, ~tvec, tvec ^ jnp.uint32(0x80000000))
            thr = lax.bitcast_convert_type(tbits, jnp.float32)

            @plsc.parallel_loop(0, nvec, unroll=8)
            def _passo(i):
                v = xbuf[pl.ds(i * _L, _L)]
                m = jnp.where(v >= thr, jnp.float32(1.0), jnp.float32(_LEAK))
                kbuf[pl.ds(i * _L, _L)] = v * m * jnp.float32(_GAIN)

            pltpu.sync_copy(kbuf, out_hbm.at[row])
            return carry_unused

        lax.fori_loop(0, rows_per_w, row_body, 0)

    return sc_topk_mask(x)
